# baseline (speedup 1.0000x reference)
import jax
import jax.numpy as jnp
from jax.experimental import pallas as pl
from jax.experimental.pallas import tpu as pltpu

B = 2
D_MODEL = 256
N_LEVELS = 4
N_HEADS = 8
N_POINTS = 4
D_FF = 1024
D_HEAD = D_MODEL // N_HEADS
SPATIAL = [(64, 64), (32, 32), (16, 16), (8, 8)]
S = sum(h * w for h, w in SPATIAL)


def _ln(x, g, b):
    m = jnp.mean(x, axis=-1, keepdims=True)
    v = jnp.mean((x - m) ** 2, axis=-1, keepdims=True)
    return (x - m) * jax.lax.rsqrt(v + 1e-5) * g + b


def _tail_kernel(x_ref, g1_ref, be1_ref, w1_ref, bf1_ref, w2_ref, bf2_ref,
                 g2_ref, be2_ref, o_ref):
    x1 = _ln(x_ref[...], g1_ref[...], be1_ref[...])
    h = jnp.maximum(x1 @ w1_ref[...] + bf1_ref[...], 0.0)
    ffn = h @ w2_ref[...] + bf2_ref[...]
    o_ref[...] = _ln(x1 + ffn, g2_ref[...], be2_ref[...])


def _tail(x, g1, be1, W1, bf1, W2, bf2, g2, be2):
    n = x.shape[0]
    blk = 640
    grid = (n // blk,)
    return pl.pallas_call(
        _tail_kernel,
        grid=grid,
        in_specs=[
            pl.BlockSpec((blk, D_MODEL), lambda i: (i, 0)),
            pl.BlockSpec((D_MODEL,), lambda i: (0,)),
            pl.BlockSpec((D_MODEL,), lambda i: (0,)),
            pl.BlockSpec((D_MODEL, D_FF), lambda i: (0, 0)),
            pl.BlockSpec((D_FF,), lambda i: (0,)),
            pl.BlockSpec((D_FF, D_MODEL), lambda i: (0, 0)),
            pl.BlockSpec((D_MODEL,), lambda i: (0,)),
            pl.BlockSpec((D_MODEL,), lambda i: (0,)),
            pl.BlockSpec((D_MODEL,), lambda i: (0,)),
        ],
        out_specs=pl.BlockSpec((blk, D_MODEL), lambda i: (i, 0)),
        out_shape=jax.ShapeDtypeStruct((n, D_MODEL), jnp.float32),
    )(x, g1, be1, W1, bf1, W2, bf2, g2, be2)


def _ms_deform_attn_core(value, sampling_locations, attention_weights):
    Bv, Sv, H, Dh = value.shape
    Lq = sampling_locations.shape[1]
    start = 0
    per_level = []
    for lid, (H_, W_) in enumerate(SPATIAL):
        v = value[:, start:start + H_ * W_]
        start += H_ * W_
        loc = sampling_locations[:, :, :, lid]
        x = loc[..., 0] * W_ - 0.5
        y = loc[..., 1] * H_ - 0.5
        x0 = jnp.floor(x)
        y0 = jnp.floor(y)
        lx = x - x0
        ly = y - y0
        x0i = x0.astype(jnp.int32)
        y0i = y0.astype(jnp.int32)

        def gather(yi, xi):
            valid = ((xi >= 0) & (xi < W_) & (yi >= 0) & (yi < H_)).astype(value.dtype)
            xc = jnp.clip(xi, 0, W_ - 1)
            yc = jnp.clip(yi, 0, H_ - 1)
            idx = yc * W_ + xc
            idx2 = jnp.transpose(idx, (0, 1, 3, 2)).reshape(Bv, Lq * N_POINTS, H)
            g = jnp.take_along_axis(v, idx2[..., None], axis=1)
            g = g.reshape(Bv, Lq, N_POINTS, H, Dh).transpose(0, 1, 3, 2, 4)
            return g * valid[..., None]

        w00 = (1 - lx) * (1 - ly)
        w01 = lx * (1 - ly)
        w10 = (1 - lx) * ly
        w11 = lx * ly
        samp = (gather(y0i, x0i) * w00[..., None]
                + gather(y0i, x0i + 1) * w01[..., None]
                + gather(y0i + 1, x0i) * w10[..., None]
                + gather(y0i + 1, x0i + 1) * w11[..., None])
        per_level.append(samp)
    out = jnp.stack(per_level, axis=3)
    out = jnp.sum(out * attention_weights[..., None], axis=(3, 4))
    return out.reshape(Bv, Lq, H * Dh)


def kernel(src, pos, reference_points, spatial_shapes, level_start_index,
           Wv, bv, Woff, boff, Wa, ba, Wout, bout,
           g1, be1, W1, bf1, W2, bf2, g2, be2):
    q = src + pos
    value = (src @ Wv + bv).reshape(B, S, N_HEADS, D_HEAD)
    offsets = (q @ Woff + boff).reshape(B, S, N_HEADS, N_LEVELS, N_POINTS, 2)
    attn = (q @ Wa + ba).reshape(B, S, N_HEADS, N_LEVELS * N_POINTS)
    attn = jax.nn.softmax(attn, axis=-1).reshape(B, S, N_HEADS, N_LEVELS, N_POINTS)
    normalizer = spatial_shapes[:, ::-1].astype(jnp.float32)
    loc = reference_points[:, :, None, :, None, :] + offsets / normalizer[None, None, None, :, None, :]
    attn_out = _ms_deform_attn_core(value, loc, attn)
    src2 = attn_out @ Wout + bout
    x = (src + src2).reshape(B * S, D_MODEL)
    return _tail(x, g1, be1, W1, bf1, W2, bf2, g2, be2).reshape(B, S, D_MODEL)


# speedup vs baseline: 40.8426x; 40.8426x over previous
import functools

import jax
import jax.numpy as jnp
import numpy as np
from jax import lax
from jax.experimental import pallas as pl
from jax.experimental.pallas import tpu as pltpu
from jax.experimental.pallas import tpu_sc as plsc

B = 2
D_MODEL = 256
N_LEVELS = 4
N_HEADS = 8
N_POINTS = 4
D_FF = 1024
D_HEAD = D_MODEL // N_HEADS
SPATIAL = [(64, 64), (32, 32), (16, 16), (8, 8)]
S = sum(h * w for h, w in SPATIAL)

A_SIZES = [(h + 1) * (w + 1) for h, w in SPATIAL]
A_STARTS = [0]
for a in A_SIZES[:-1]:
    A_STARTS.append(A_STARTS[-1] + a)
A_TOT = sum(A_SIZES)

N_OUT = B * S * N_HEADS
N_SAMP = N_LEVELS * N_POINTS
NW = 32
QPW = N_OUT // NW
CQ = 8
NCHUNK = QPW // CQ
DPC = CQ * N_SAMP


def _proj_kernel(src_ref, pos_ref, wv_ref, bv_ref, wcat_ref, bcat_ref,
                 bd_ref, val_ref, off_ref, attn_ref):
    s = src_ref[0]
    q = s + pos_ref[0]
    val_ref[0] = jnp.dot(s, wv_ref[...], preferred_element_type=jnp.float32) + bv_ref[...]
    oa = jnp.dot(q, wcat_ref[...], preferred_element_type=jnp.float32) + bcat_ref[...]
    off_ref[0] = oa[:, :D_MODEL]
    e = jnp.exp(oa[:, D_MODEL:])
    denom = jnp.dot(e, bd_ref[...], preferred_element_type=jnp.float32)
    attn_ref[0] = e / denom


def _proj(src, pos, Wv, bv, Wcat, bcat, bd):
    s_blk = 680
    grid = (B, S // s_blk)
    return pl.pallas_call(
        _proj_kernel,
        grid=grid,
        in_specs=[
            pl.BlockSpec((1, s_blk, D_MODEL), lambda b, i: (b, i, 0)),
            pl.BlockSpec((1, s_blk, D_MODEL), lambda b, i: (b, i, 0)),
            pl.BlockSpec((D_MODEL, D_MODEL), lambda b, i: (0, 0)),
            pl.BlockSpec((1, D_MODEL), lambda b, i: (0, 0)),
            pl.BlockSpec((D_MODEL, 384), lambda b, i: (0, 0)),
            pl.BlockSpec((1, 384), lambda b, i: (0, 0)),
            pl.BlockSpec((128, 128), lambda b, i: (0, 0)),
        ],
        out_specs=[
            pl.BlockSpec((1, s_blk, D_MODEL), lambda b, i: (b, i, 0)),
            pl.BlockSpec((1, s_blk, D_MODEL), lambda b, i: (b, i, 0)),
            pl.BlockSpec((1, s_blk, 128), lambda b, i: (b, i, 0)),
        ],
        out_shape=[
            jax.ShapeDtypeStruct((B, S, D_MODEL), jnp.float32),
            jax.ShapeDtypeStruct((B, S, D_MODEL), jnp.float32),
            jax.ShapeDtypeStruct((B, S, 128), jnp.float32),
        ],
    )(src, pos, Wv, bv, Wcat, bcat, bd)


def _bcast16(v, lane):
    idx = jnp.full((16,), lane, dtype=jnp.int32)
    return v.at[idx].get(mode="promise_in_bounds")


def _sc_sample(table, idx_h, w_h):
    mesh = plsc.VectorSubcoreMesh(core_axis_name="c", subcore_axis_name="s")

    @functools.partial(
        pl.kernel,
        mesh=mesh,
        out_type=jax.ShapeDtypeStruct((N_OUT, D_HEAD), jnp.float32),
        scratch_types=[
            pltpu.VMEM((1, DPC), jnp.int32),
            pltpu.VMEM((CQ, 64), jnp.float32),
            pltpu.VMEM((DPC, 128), jnp.float32),
            pltpu.VMEM((CQ, D_HEAD), jnp.float32),
            pltpu.SemaphoreType.DMA,
        ],
    )
    def body(table_ref, idx_ref, w_ref, out_ref, idx_v, w_v, rows_v, out_v, sem):
        wid = lax.axis_index("s") * 2 + lax.axis_index("c")

        def chunk_body(ci, carry):
            qb = pl.multiple_of(wid * QPW + ci * CQ, CQ)
            irow = wid * NCHUNK + ci
            pltpu.sync_copy(idx_ref.at[pl.ds(irow, 1), :], idx_v)
            pltpu.sync_copy(w_ref.at[pl.ds(qb, CQ), :], w_v)
            pltpu.async_copy(table_ref.at[idx_v.at[0]], rows_v, sem).wait()

            def q_body(j, c2):
                acc0 = jnp.zeros((16,), jnp.float32)
                acc1 = jnp.zeros((16,), jnp.float32)
                rbase = j * N_SAMP
                for kk in range(4):
                    wv = w_v[j, pl.ds(kk * 16, 16)]
                    for k2 in range(16):
                        t = kk * 4 + k2 // 4
                        off = (k2 % 4) * 32
                        wb = _bcast16(wv, k2)
                        acc0 = acc0 + wb * rows_v[rbase + t, pl.ds(off, 16)]
                        acc1 = acc1 + wb * rows_v[rbase + t, pl.ds(off + 16, 16)]
                out_v[j, pl.ds(0, 16)] = acc0
                out_v[j, pl.ds(16, 16)] = acc1
                return c2

            lax.fori_loop(0, CQ, q_body, 0, unroll=False)
            pltpu.sync_copy(out_v, out_ref.at[pl.ds(qb, CQ), :])
            return carry

        lax.fori_loop(0, NCHUNK, chunk_body, 0, unroll=False)

    return body(table, idx_h, w_h)


def _ln(x, g, b):
    m = jnp.mean(x, axis=-1, keepdims=True)
    v = jnp.mean((x - m) ** 2, axis=-1, keepdims=True)
    return (x - m) * lax.rsqrt(v + 1e-5) * g + b


def _tail_kernel(ao_ref, src_ref, wout_ref, bout_ref, g1_ref, be1_ref,
                 w1_ref, bf1_ref, w2_ref, bf2_ref, g2_ref, be2_ref, o_ref):
    x = src_ref[...] + jnp.dot(ao_ref[...], wout_ref[...],
                               preferred_element_type=jnp.float32) + bout_ref[...]
    x1 = _ln(x, g1_ref[...], be1_ref[...])
    h = jnp.maximum(jnp.dot(x1, w1_ref[...], preferred_element_type=jnp.float32)
                    + bf1_ref[...], 0.0)
    ffn = jnp.dot(h, w2_ref[...], preferred_element_type=jnp.float32) + bf2_ref[...]
    o_ref[...] = _ln(x1 + ffn, g2_ref[...], be2_ref[...])


def _tail(ao, x, Wout, bout, g1, be1, W1, bf1, W2, bf2, g2, be2):
    n = x.shape[0]
    blk = 640
    grid = (n // blk,)
    vec = lambda d: pl.BlockSpec((1, d), lambda i: (0, 0))
    return pl.pallas_call(
        _tail_kernel,
        grid=grid,
        in_specs=[
            pl.BlockSpec((blk, D_MODEL), lambda i: (i, 0)),
            pl.BlockSpec((blk, D_MODEL), lambda i: (i, 0)),
            pl.BlockSpec((D_MODEL, D_MODEL), lambda i: (0, 0)),
            vec(D_MODEL), vec(D_MODEL), vec(D_MODEL),
            pl.BlockSpec((D_MODEL, D_FF), lambda i: (0, 0)),
            vec(D_FF),
            pl.BlockSpec((D_FF, D_MODEL), lambda i: (0, 0)),
            vec(D_MODEL), vec(D_MODEL), vec(D_MODEL),
        ],
        out_specs=pl.BlockSpec((blk, D_MODEL), lambda i: (i, 0)),
        out_shape=jax.ShapeDtypeStruct((n, D_MODEL), jnp.float32),
    )(ao, x, Wout, bout.reshape(1, -1), g1.reshape(1, -1), be1.reshape(1, -1),
      W1, bf1.reshape(1, -1), W2, bf2.reshape(1, -1), g2.reshape(1, -1),
      be2.reshape(1, -1))


def _build_anchor_table(value):
    vt = value.reshape(B, S, N_HEADS, D_HEAD).transpose(0, 2, 1, 3)
    pieces = []
    start = 0
    for (H_, W_) in SPATIAL:
        g = vt[:, :, start:start + H_ * W_].reshape(B, N_HEADS, H_, W_, D_HEAD)
        gp = jnp.pad(g, ((0, 0), (0, 0), (1, 1), (1, 1), (0, 0)))
        a00 = gp[:, :, 0:H_ + 1, 0:W_ + 1]
        a10 = gp[:, :, 1:H_ + 2, 0:W_ + 1]
        a01 = gp[:, :, 0:H_ + 1, 1:W_ + 2]
        a11 = gp[:, :, 1:H_ + 2, 1:W_ + 2]
        anch = jnp.concatenate([a00, a10, a01, a11], axis=-1)
        pieces.append(anch.reshape(B, N_HEADS, (H_ + 1) * (W_ + 1), 128))
        start += H_ * W_
    return jnp.concatenate(pieces, axis=2).reshape(B * N_HEADS * A_TOT, 128)


def kernel(src, pos, reference_points, spatial_shapes, level_start_index,
           Wv, bv, Woff, boff, Wa, ba, Wout, bout,
           g1, be1, W1, bf1, W2, bf2, g2, be2):
    Wcat = jnp.concatenate([Woff, Wa], axis=1)
    bcat = jnp.concatenate([boff, ba], axis=0).reshape(1, 384)
    bd = jnp.asarray(np.kron(np.eye(N_HEADS, dtype=np.float32),
                             np.ones((16, 16), dtype=np.float32)))
    value, off, attn = _proj(src, pos, Wv, bv.reshape(1, -1), Wcat, bcat, bd)

    table = _build_anchor_table(value)

    offsets = off.reshape(B, S, N_HEADS, N_LEVELS, N_POINTS, 2)
    attnw = attn.reshape(B, S, N_HEADS, N_LEVELS, N_POINTS)
    norm = np.array([(w, h) for h, w in SPATIAL], dtype=np.float32)
    loc = (reference_points[:, :, None, :, None, :]
           + offsets / norm[None, None, None, :, None, :])

    idx_levels, w_levels = [], []
    for lid, (H_, W_) in enumerate(SPATIAL):
        locl = loc[:, :, :, lid]
        x = locl[..., 0] * W_ - 0.5
        y = locl[..., 1] * H_ - 0.5
        x0f = jnp.floor(x)
        y0f = jnp.floor(y)
        lx = x - x0f
        ly = y - y0f
        x0 = x0f.astype(jnp.int32)
        y0 = y0f.astype(jnp.int32)
        x0c = jnp.clip(x0, -1, W_ - 1)
        y0c = jnp.clip(y0, -1, H_ - 1)
        idx_levels.append(A_STARTS[lid] + (y0c + 1) * (W_ + 1) + (x0c + 1))
        wgts = []
        for yi, xi, wgt in ((y0, x0, (1 - lx) * (1 - ly)),
                            (y0 + 1, x0, (1 - lx) * ly),
                            (y0, x0 + 1, lx * (1 - ly)),
                            (y0 + 1, x0 + 1, lx * ly)):
            valid = (xi >= 0) & (xi < W_) & (yi >= 0) & (yi < H_)
            wgts.append(jnp.where(valid, wgt, 0.0))
        w_levels.append(jnp.stack(wgts, -1))

    aidx = jnp.stack(idx_levels, 3)
    w4 = jnp.stack(w_levels, 3) * attnw[..., None]

    bidx = jnp.arange(B, dtype=jnp.int32).reshape(B, 1, 1, 1, 1)
    hidx = jnp.arange(N_HEADS, dtype=jnp.int32).reshape(1, 1, N_HEADS, 1, 1)
    gidx = (bidx * N_HEADS + hidx) * A_TOT + aidx

    idx_h = gidx.reshape(N_OUT * N_SAMP // DPC, DPC)
    w_h = w4.reshape(N_OUT, 64)

    attn_out = _sc_sample(table, idx_h, w_h)

    ao = attn_out.reshape(B * S, D_MODEL)
    out = _tail(ao, src.reshape(B * S, D_MODEL), Wout, bout,
                g1, be1, W1, bf1, W2, bf2, g2, be2)
    return out.reshape(B, S, D_MODEL)


# speedup vs baseline: 56.7504x; 1.3895x over previous
import functools

import jax
import jax.numpy as jnp
import numpy as np
from jax import lax
from jax.experimental import pallas as pl
from jax.experimental.pallas import tpu as pltpu
from jax.experimental.pallas import tpu_sc as plsc

B = 2
D_MODEL = 256
N_LEVELS = 4
N_HEADS = 8
N_POINTS = 4
D_FF = 1024
D_HEAD = D_MODEL // N_HEADS
SPATIAL = [(64, 64), (32, 32), (16, 16), (8, 8)]
S = sum(h * w for h, w in SPATIAL)

A_SIZES = [(h + 1) * (w + 1) for h, w in SPATIAL]
A_STARTS = [0]
for a in A_SIZES[:-1]:
    A_STARTS.append(A_STARTS[-1] + a)
A_TOT = sum(A_SIZES)

N_OUT = B * S * N_HEADS
N_SAMP = N_LEVELS * N_POINTS
NW = 32
QPW = N_OUT // NW
CQ = 16
NCHUNK = QPW // CQ
DPC = CQ * N_SAMP
GPC = DPC // 128
SUP = 10
NSUP = NCHUNK // SUP
SQ = SUP * CQ
IR = SUP * GPC
IRP = 24


def _proj_kernel(src_ref, pos_ref, wv_ref, bv_ref, wcat_ref, bcat_ref,
                 bd_ref, val_ref, off_ref, attn_ref):
    s = src_ref[0]
    q = s + pos_ref[0]
    val_ref[0] = jnp.dot(s, wv_ref[...], preferred_element_type=jnp.float32) + bv_ref[...]
    oa = jnp.dot(q, wcat_ref[...], preferred_element_type=jnp.float32) + bcat_ref[...]
    off_ref[0] = oa[:, :D_MODEL]
    e = jnp.exp(oa[:, D_MODEL:])
    denom = jnp.dot(e, bd_ref[...], preferred_element_type=jnp.float32)
    attn_ref[0] = e / denom


def _proj(src, pos, Wv, bv, Wcat, bcat, bd):
    s_blk = 680
    grid = (B, S // s_blk)
    return pl.pallas_call(
        _proj_kernel,
        grid=grid,
        in_specs=[
            pl.BlockSpec((1, s_blk, D_MODEL), lambda b, i: (b, i, 0)),
            pl.BlockSpec((1, s_blk, D_MODEL), lambda b, i: (b, i, 0)),
            pl.BlockSpec((D_MODEL, D_MODEL), lambda b, i: (0, 0)),
            pl.BlockSpec((1, D_MODEL), lambda b, i: (0, 0)),
            pl.BlockSpec((D_MODEL, 384), lambda b, i: (0, 0)),
            pl.BlockSpec((1, 384), lambda b, i: (0, 0)),
            pl.BlockSpec((128, 128), lambda b, i: (0, 0)),
        ],
        out_specs=[
            pl.BlockSpec((1, s_blk, D_MODEL), lambda b, i: (b, i, 0)),
            pl.BlockSpec((1, s_blk, D_MODEL), lambda b, i: (b, i, 0)),
            pl.BlockSpec((1, s_blk, 128), lambda b, i: (b, i, 0)),
        ],
        out_shape=[
            jax.ShapeDtypeStruct((B, S, D_MODEL), jnp.float32),
            jax.ShapeDtypeStruct((B, S, D_MODEL), jnp.float32),
            jax.ShapeDtypeStruct((B, S, 128), jnp.float32),
        ],
    )(src, pos, Wv, bv, Wcat, bcat, bd)


def _bcast16(v, lane):
    idx = jnp.full((16,), lane, dtype=jnp.int32)
    return v.at[idx].get(mode="promise_in_bounds")


def _sc_sample(table, idx_h, w_h):
    mesh = plsc.VectorSubcoreMesh(core_axis_name="c", subcore_axis_name="s")

    @functools.partial(
        pl.kernel,
        mesh=mesh,
        out_type=jax.ShapeDtypeStruct((N_OUT, D_HEAD), jnp.float32),
        scratch_types=[
            pltpu.VMEM((IRP, 128), jnp.int32),
            pltpu.VMEM((SQ, 64), jnp.float32),
            pltpu.VMEM((DPC, 128), jnp.float32),
            pltpu.VMEM((DPC, 128), jnp.float32),
            pltpu.VMEM((SQ, D_HEAD), jnp.float32),
            pltpu.SemaphoreType.DMA,
            pltpu.SemaphoreType.DMA,
        ],
    )
    def body(table_ref, idx_ref, w_ref, out_ref, idx_v, w_v, rows0, rows1,
             out_v, sem0, sem1):
        wid = lax.axis_index("s") * 2 + lax.axis_index("c")
        bufs = (rows0, rows1)
        sems = (sem0, sem1)

        def fire(c):
            buf, sem = bufs[c % 2], sems[c % 2]
            return [
                pltpu.async_copy(
                    table_ref.at[idx_v.at[c * GPC + r]],
                    buf.at[pl.ds(r * 128, 128), :],
                    sem,
                )
                for r in range(GPC)
            ]

        def sup_body(g, carry):
            qb = pl.multiple_of(wid * QPW + g * SQ, CQ)
            pltpu.sync_copy(idx_ref.at[wid, g], idx_v)
            pltpu.sync_copy(w_ref.at[pl.ds(qb, SQ), :], w_v)
            descs = fire(0)
            for c in range(SUP):
                buf = bufs[c % 2]
                nxt = fire(c + 1) if c + 1 < SUP else []
                for d in descs:
                    d.wait()
                descs = nxt

                def q_body(j, c2, _c=c, _buf=buf):
                    acc0 = jnp.zeros((16,), jnp.float32)
                    acc1 = jnp.zeros((16,), jnp.float32)
                    rbase = j * N_SAMP
                    qrow = _c * CQ + j
                    for kk in range(4):
                        wv = w_v[qrow, pl.ds(kk * 16, 16)]
                        for k2 in range(16):
                            t = kk * 4 + k2 // 4
                            off = (k2 % 4) * 32
                            wb = _bcast16(wv, k2)
                            acc0 = acc0 + wb * _buf[rbase + t, pl.ds(off, 16)]
                            acc1 = acc1 + wb * _buf[rbase + t, pl.ds(off + 16, 16)]
                    out_v[qrow, pl.ds(0, 16)] = acc0
                    out_v[qrow, pl.ds(16, 16)] = acc1
                    return c2

                lax.fori_loop(0, CQ, q_body, 0, unroll=False)
            pltpu.sync_copy(out_v, out_ref.at[pl.ds(qb, SQ), :])
            return carry

        lax.fori_loop(0, NSUP, sup_body, 0, unroll=False)

    return body(table, idx_h, w_h)


def _ln(x, g, b):
    m = jnp.mean(x, axis=-1, keepdims=True)
    v = jnp.mean((x - m) ** 2, axis=-1, keepdims=True)
    return (x - m) * lax.rsqrt(v + 1e-5) * g + b


def _tail_kernel(ao_ref, src_ref, wout_ref, bout_ref, g1_ref, be1_ref,
                 w1_ref, bf1_ref, w2_ref, bf2_ref, g2_ref, be2_ref, o_ref):
    x = src_ref[...] + jnp.dot(ao_ref[...], wout_ref[...],
                               preferred_element_type=jnp.float32) + bout_ref[...]
    x1 = _ln(x, g1_ref[...], be1_ref[...])
    h = jnp.maximum(jnp.dot(x1, w1_ref[...], preferred_element_type=jnp.float32)
                    + bf1_ref[...], 0.0)
    ffn = jnp.dot(h, w2_ref[...], preferred_element_type=jnp.float32) + bf2_ref[...]
    o_ref[...] = _ln(x1 + ffn, g2_ref[...], be2_ref[...])


def _tail(ao, x, Wout, bout, g1, be1, W1, bf1, W2, bf2, g2, be2):
    n = x.shape[0]
    blk = 640
    grid = (n // blk,)
    vec = lambda d: pl.BlockSpec((1, d), lambda i: (0, 0))
    return pl.pallas_call(
        _tail_kernel,
        grid=grid,
        in_specs=[
            pl.BlockSpec((blk, D_MODEL), lambda i: (i, 0)),
            pl.BlockSpec((blk, D_MODEL), lambda i: (i, 0)),
            pl.BlockSpec((D_MODEL, D_MODEL), lambda i: (0, 0)),
            vec(D_MODEL), vec(D_MODEL), vec(D_MODEL),
            pl.BlockSpec((D_MODEL, D_FF), lambda i: (0, 0)),
            vec(D_FF),
            pl.BlockSpec((D_FF, D_MODEL), lambda i: (0, 0)),
            vec(D_MODEL), vec(D_MODEL), vec(D_MODEL),
        ],
        out_specs=pl.BlockSpec((blk, D_MODEL), lambda i: (i, 0)),
        out_shape=jax.ShapeDtypeStruct((n, D_MODEL), jnp.float32),
    )(ao, x, Wout, bout.reshape(1, -1), g1.reshape(1, -1), be1.reshape(1, -1),
      W1, bf1.reshape(1, -1), W2, bf2.reshape(1, -1), g2.reshape(1, -1),
      be2.reshape(1, -1))


def _build_anchor_table(value):
    vt = value.reshape(B, S, N_HEADS, D_HEAD).transpose(0, 2, 1, 3)
    pieces = []
    start = 0
    for (H_, W_) in SPATIAL:
        g = vt[:, :, start:start + H_ * W_].reshape(B, N_HEADS, H_, W_, D_HEAD)
        gp = jnp.pad(g, ((0, 0), (0, 0), (1, 1), (1, 1), (0, 0)))
        a00 = gp[:, :, 0:H_ + 1, 0:W_ + 1]
        a10 = gp[:, :, 1:H_ + 2, 0:W_ + 1]
        a01 = gp[:, :, 0:H_ + 1, 1:W_ + 2]
        a11 = gp[:, :, 1:H_ + 2, 1:W_ + 2]
        anch = jnp.concatenate([a00, a10, a01, a11], axis=-1)
        pieces.append(anch.reshape(B, N_HEADS, (H_ + 1) * (W_ + 1), 128))
        start += H_ * W_
    return jnp.concatenate(pieces, axis=2).reshape(B * N_HEADS * A_TOT, 128)


def kernel(src, pos, reference_points, spatial_shapes, level_start_index,
           Wv, bv, Woff, boff, Wa, ba, Wout, bout,
           g1, be1, W1, bf1, W2, bf2, g2, be2):
    Wcat = jnp.concatenate([Woff, Wa], axis=1)
    bcat = jnp.concatenate([boff, ba], axis=0).reshape(1, 384)
    bd = jnp.asarray(np.kron(np.eye(N_HEADS, dtype=np.float32),
                             np.ones((16, 16), dtype=np.float32)))
    value, off, attn = _proj(src, pos, Wv, bv.reshape(1, -1), Wcat, bcat, bd)

    table = _build_anchor_table(value)

    offsets = off.reshape(B, S, N_HEADS, N_LEVELS, N_POINTS, 2)
    attnw = attn.reshape(B, S, N_HEADS, N_LEVELS, N_POINTS)
    norm = np.array([(w, h) for h, w in SPATIAL], dtype=np.float32)
    loc = (reference_points[:, :, None, :, None, :]
           + offsets / norm[None, None, None, :, None, :])

    idx_levels, w_levels = [], []
    for lid, (H_, W_) in enumerate(SPATIAL):
        locl = loc[:, :, :, lid]
        x = locl[..., 0] * W_ - 0.5
        y = locl[..., 1] * H_ - 0.5
        x0f = jnp.floor(x)
        y0f = jnp.floor(y)
        lx = x - x0f
        ly = y - y0f
        x0 = x0f.astype(jnp.int32)
        y0 = y0f.astype(jnp.int32)
        x0c = jnp.clip(x0, -1, W_ - 1)
        y0c = jnp.clip(y0, -1, H_ - 1)
        idx_levels.append(A_STARTS[lid] + (y0c + 1) * (W_ + 1) + (x0c + 1))
        wgts = []
        for yi, xi, wgt in ((y0, x0, (1 - lx) * (1 - ly)),
                            (y0 + 1, x0, (1 - lx) * ly),
                            (y0, x0 + 1, lx * (1 - ly)),
                            (y0 + 1, x0 + 1, lx * ly)):
            valid = (xi >= 0) & (xi < W_) & (yi >= 0) & (yi < H_)
            wgts.append(jnp.where(valid, wgt, 0.0))
        w_levels.append(jnp.stack(wgts, -1))

    aidx = jnp.stack(idx_levels, 3)
    w4 = jnp.stack(w_levels, 3) * attnw[..., None]

    bidx = jnp.arange(B, dtype=jnp.int32).reshape(B, 1, 1, 1, 1)
    hidx = jnp.arange(N_HEADS, dtype=jnp.int32).reshape(1, 1, N_HEADS, 1, 1)
    gidx = (bidx * N_HEADS + hidx) * A_TOT + aidx

    idx_h = gidx.reshape(NW, NSUP, IR, 128)
    idx_h = jnp.pad(idx_h, ((0, 0), (0, 0), (0, IRP - IR), (0, 0)))
    w_h = w4.reshape(N_OUT, 64)

    attn_out = _sc_sample(table, idx_h, w_h)

    ao = attn_out.reshape(B * S, D_MODEL)
    out = _tail(ao, src.reshape(B * S, D_MODEL), Wout, bout,
                g1, be1, W1, bf1, W2, bf2, g2, be2)
    return out.reshape(B, S, D_MODEL)


# speedup vs baseline: 80.9347x; 1.4262x over previous
import functools

import jax
import jax.numpy as jnp
import numpy as np
from jax import lax
from jax.experimental import pallas as pl
from jax.experimental.pallas import tpu as pltpu
from jax.experimental.pallas import tpu_sc as plsc

B = 2
D_MODEL = 256
N_LEVELS = 4
N_HEADS = 8
N_POINTS = 4
D_FF = 1024
D_HEAD = D_MODEL // N_HEADS
SPATIAL = [(64, 64), (32, 32), (16, 16), (8, 8)]
S = sum(h * w for h, w in SPATIAL)

A_SIZES = [(h + 1) * (w + 1) for h, w in SPATIAL]
A_STARTS = [0]
for a in A_SIZES[:-1]:
    A_STARTS.append(A_STARTS[-1] + a)
A_TOT = sum(A_SIZES)

N_OUT = B * S * N_HEADS
N_SAMP = N_LEVELS * N_POINTS
NW = 32
QPW = N_OUT // NW
CQ = 16
NCHUNK = QPW // CQ
DPC = CQ * N_SAMP
GPC = DPC // 128
SUP = 10
NSUP = NCHUNK // SUP
SQ = SUP * CQ
IR = SUP * GPC
IRP = 24


_L_OF = np.arange(128) // 4 % 4
_H_OF = np.arange(128) // 16
_WF = np.array([SPATIAL[l][1] for l in _L_OF], np.float32)
_HF = np.array([SPATIAL[l][0] for l in _L_OF], np.float32)
_ABASE = np.array([A_STARTS[l] for l in _L_OF], np.int32) + _H_OF.astype(np.int32) * A_TOT
_WP1 = (_WF + 1).astype(np.int32)


def _proj_kernel(src_ref, pos_ref, rxy_ref, wv_ref, bv_ref, wcat_ref, bcat_ref,
                 bd_ref, ex_ref, ey_ref, cf_ref, ci_ref,
                 val_ref, idx_ref, w0_ref, w1_ref, w2_ref, w3_ref):
    s = src_ref[0]
    q = s + pos_ref[0]
    val_ref[0] = jnp.dot(s, wv_ref[...], preferred_element_type=jnp.float32) + bv_ref[...]
    oa = jnp.dot(q, wcat_ref[...], preferred_element_type=jnp.float32) + bcat_ref[...]
    offx = oa[:, 0:128]
    offy = oa[:, 128:256]
    e = jnp.exp(oa[:, 256:384])
    denom = jnp.dot(e, bd_ref[...], preferred_element_type=jnp.float32,
                    precision=lax.Precision.HIGHEST)
    attnw = e / denom

    rxy = rxy_ref[0]
    rx = jnp.dot(rxy, ex_ref[...], preferred_element_type=jnp.float32,
                 precision=lax.Precision.HIGHEST)
    ry = jnp.dot(rxy, ey_ref[...], preferred_element_type=jnp.float32,
                 precision=lax.Precision.HIGHEST)

    cWf = cf_ref[0, :]
    cHf = cf_ref[1, :]
    cinvW = cf_ref[2, :]
    cinvH = cf_ref[3, :]
    cWi = ci_ref[0, :]
    cHi = ci_ref[1, :]
    cWp1 = ci_ref[2, :]
    cAbase = ci_ref[3, :]

    x = (rx + offx * cinvW) * cWf - 0.5
    y = (ry + offy * cinvH) * cHf - 0.5
    x0f = jnp.floor(x)
    y0f = jnp.floor(y)
    lx = x - x0f
    ly = y - y0f
    x0 = x0f.astype(jnp.int32)
    y0 = y0f.astype(jnp.int32)
    x0c = jnp.clip(x0, -1, cWi - 1)
    y0c = jnp.clip(y0, -1, cHi - 1)
    b = pl.program_id(0)
    idx_ref[0] = cAbase + (y0c + 1) * cWp1 + (x0c + 1) + b * (N_HEADS * A_TOT)

    vx0 = (x0 >= 0) & (x0 < cWi)
    vx1 = (x0 >= -1) & (x0 + 1 < cWi)
    vy0 = (y0 >= 0) & (y0 < cHi)
    vy1 = (y0 >= -1) & (y0 + 1 < cHi)
    omlx = 1.0 - lx
    omly = 1.0 - ly
    zero = jnp.zeros_like(lx)
    w0_ref[0] = jnp.where(vx0 & vy0, omlx * omly, zero) * attnw
    w1_ref[0] = jnp.where(vx0 & vy1, omlx * ly, zero) * attnw
    w2_ref[0] = jnp.where(vx1 & vy0, lx * omly, zero) * attnw
    w3_ref[0] = jnp.where(vx1 & vy1, lx * ly, zero) * attnw


def _proj(src, pos, rxy, Wv, bv, Wcat, bcat, bd, ex, ey, cf, ci):
    s_blk = 680
    grid = (B, S // s_blk)
    blk = lambda d: pl.BlockSpec((1, s_blk, d), lambda b, i: (b, i, 0))
    full = lambda r, c: pl.BlockSpec((r, c), lambda b, i: (0, 0))
    f32 = jnp.float32
    return pl.pallas_call(
        _proj_kernel,
        grid=grid,
        in_specs=[
            blk(D_MODEL), blk(D_MODEL), blk(8),
            full(D_MODEL, D_MODEL), full(1, D_MODEL),
            full(D_MODEL, 384), full(1, 384),
            full(128, 128), full(8, 128), full(8, 128),
            full(4, 128), full(4, 128),
        ],
        out_specs=[blk(D_MODEL), blk(128), blk(128), blk(128), blk(128), blk(128)],
        out_shape=[
            jax.ShapeDtypeStruct((B, S, D_MODEL), f32),
            jax.ShapeDtypeStruct((B, S, 128), jnp.int32),
            jax.ShapeDtypeStruct((B, S, 128), f32),
            jax.ShapeDtypeStruct((B, S, 128), f32),
            jax.ShapeDtypeStruct((B, S, 128), f32),
            jax.ShapeDtypeStruct((B, S, 128), f32),
        ],
    )(src, pos, rxy, Wv, bv, Wcat, bcat, bd, ex, ey, cf, ci)


def _bcast16(v, lane):
    idx = jnp.full((16,), lane, dtype=jnp.int32)
    return v.at[idx].get(mode="promise_in_bounds")


def _sc_sample(table, idx_h, w_h):
    mesh = plsc.VectorSubcoreMesh(core_axis_name="c", subcore_axis_name="s")

    @functools.partial(
        pl.kernel,
        mesh=mesh,
        out_type=jax.ShapeDtypeStruct((N_OUT, D_HEAD), jnp.float32),
        scratch_types=[
            pltpu.VMEM((IRP, 128), jnp.int32),
            pltpu.VMEM((SQ, 64), jnp.float32),
            pltpu.VMEM((DPC, 128), jnp.float32),
            pltpu.VMEM((DPC, 128), jnp.float32),
            pltpu.VMEM((SQ, D_HEAD), jnp.float32),
            pltpu.SemaphoreType.DMA,
            pltpu.SemaphoreType.DMA,
        ],
    )
    def body(table_ref, idx_ref, w_ref, out_ref,
             idx_v, w_v, rows0, rows1, out_v, sem0, sem1):
        wid = lax.axis_index("s") * 2 + lax.axis_index("c")
        bufs = (rows0, rows1)
        sems = (sem0, sem1)

        def fire(c):
            buf, sem = bufs[c % 2], sems[c % 2]
            return [
                pltpu.async_copy(
                    table_ref.at[idx_v.at[c * GPC + r]],
                    buf.at[pl.ds(r * 128, 128), :],
                    sem,
                )
                for r in range(GPC)
            ]

        def sup_body(g, carry):
            qb = pl.multiple_of(wid * QPW + g * SQ, CQ)
            pltpu.sync_copy(idx_ref.at[wid, g], idx_v)
            pltpu.sync_copy(w_ref.at[pl.ds(qb, SQ), :], w_v)
            descs = fire(0)
            for c in range(SUP):
                buf = bufs[c % 2]
                nxt = fire(c + 1) if c + 1 < SUP else []
                for d in descs:
                    d.wait()
                descs = nxt

                def q_body(j, c2, _c=c, _buf=buf):
                    acc0 = jnp.zeros((16,), jnp.float32)
                    acc1 = jnp.zeros((16,), jnp.float32)
                    rbase = j * N_SAMP
                    qrow = _c * CQ + j
                    for cc in range(4):
                        wv = w_v[qrow, pl.ds(cc * 16, 16)]
                        for t in range(16):
                            wb = _bcast16(wv, t)
                            acc0 = acc0 + wb * _buf[rbase + t, pl.ds(cc * 32, 16)]
                            acc1 = acc1 + wb * _buf[rbase + t, pl.ds(cc * 32 + 16, 16)]
                    out_v[qrow, pl.ds(0, 16)] = acc0
                    out_v[qrow, pl.ds(16, 16)] = acc1
                    return c2

                lax.fori_loop(0, CQ, q_body, 0, unroll=False)
            pltpu.sync_copy(out_v, out_ref.at[pl.ds(qb, SQ), :])
            return carry

        lax.fori_loop(0, NSUP, sup_body, 0, unroll=False)

    return body(table, idx_h, w_h)


def _ln(x, g, b):
    m = jnp.mean(x, axis=-1, keepdims=True)
    v = jnp.mean((x - m) ** 2, axis=-1, keepdims=True)
    return (x - m) * lax.rsqrt(v + 1e-5) * g + b


def _tail_kernel(ao_ref, src_ref, wout_ref, bout_ref, g1_ref, be1_ref,
                 w1_ref, bf1_ref, w2_ref, bf2_ref, g2_ref, be2_ref, o_ref):
    x = src_ref[...] + jnp.dot(ao_ref[...], wout_ref[...],
                               preferred_element_type=jnp.float32) + bout_ref[...]
    x1 = _ln(x, g1_ref[...], be1_ref[...])
    h = jnp.maximum(jnp.dot(x1, w1_ref[...], preferred_element_type=jnp.float32)
                    + bf1_ref[...], 0.0)
    ffn = jnp.dot(h, w2_ref[...], preferred_element_type=jnp.float32) + bf2_ref[...]
    o_ref[...] = _ln(x1 + ffn, g2_ref[...], be2_ref[...])


def _tail(ao, x, Wout, bout, g1, be1, W1, bf1, W2, bf2, g2, be2):
    n = x.shape[0]
    blk = 640
    grid = (n // blk,)
    vec = lambda d: pl.BlockSpec((1, d), lambda i: (0, 0))
    return pl.pallas_call(
        _tail_kernel,
        grid=grid,
        in_specs=[
            pl.BlockSpec((blk, D_MODEL), lambda i: (i, 0)),
            pl.BlockSpec((blk, D_MODEL), lambda i: (i, 0)),
            pl.BlockSpec((D_MODEL, D_MODEL), lambda i: (0, 0)),
            vec(D_MODEL), vec(D_MODEL), vec(D_MODEL),
            pl.BlockSpec((D_MODEL, D_FF), lambda i: (0, 0)),
            vec(D_FF),
            pl.BlockSpec((D_FF, D_MODEL), lambda i: (0, 0)),
            vec(D_MODEL), vec(D_MODEL), vec(D_MODEL),
        ],
        out_specs=pl.BlockSpec((blk, D_MODEL), lambda i: (i, 0)),
        out_shape=jax.ShapeDtypeStruct((n, D_MODEL), jnp.float32),
    )(ao, x, Wout, bout.reshape(1, -1), g1.reshape(1, -1), be1.reshape(1, -1),
      W1, bf1.reshape(1, -1), W2, bf2.reshape(1, -1), g2.reshape(1, -1),
      be2.reshape(1, -1))


def _build_anchor_table(value):
    vt = value.reshape(B, S, N_HEADS, D_HEAD).transpose(0, 2, 1, 3)
    pieces = []
    start = 0
    for (H_, W_) in SPATIAL:
        g = vt[:, :, start:start + H_ * W_].reshape(B, N_HEADS, H_, W_, D_HEAD)
        gp = jnp.pad(g, ((0, 0), (0, 0), (1, 1), (1, 1), (0, 0)))
        a00 = gp[:, :, 0:H_ + 1, 0:W_ + 1]
        a10 = gp[:, :, 1:H_ + 2, 0:W_ + 1]
        a01 = gp[:, :, 0:H_ + 1, 1:W_ + 2]
        a11 = gp[:, :, 1:H_ + 2, 1:W_ + 2]
        anch = jnp.concatenate([a00, a10, a01, a11], axis=-1)
        pieces.append(anch.reshape(B, N_HEADS, (H_ + 1) * (W_ + 1), 128))
        start += H_ * W_
    return jnp.concatenate(pieces, axis=2).reshape(B * N_HEADS * A_TOT, 128)


def kernel(src, pos, reference_points, spatial_shapes, level_start_index,
           Wv, bv, Woff, boff, Wa, ba, Wout, bout,
           g1, be1, W1, bf1, W2, bf2, g2, be2):
    Wcat = jnp.concatenate([Woff[:, 0::2], Woff[:, 1::2], Wa], axis=1)
    bcat = jnp.concatenate([boff[0::2], boff[1::2], ba], axis=0).reshape(1, 384)
    bd = jnp.asarray(np.kron(np.eye(N_HEADS, dtype=np.float32),
                             np.ones((16, 16), dtype=np.float32)))
    ex = np.zeros((8, 128), np.float32)
    ey = np.zeros((8, 128), np.float32)
    ex[2 * _L_OF, np.arange(128)] = 1.0
    ey[2 * _L_OF + 1, np.arange(128)] = 1.0
    cf = np.stack([_WF, _HF, 1.0 / _WF, 1.0 / _HF]).astype(np.float32)
    ci = np.stack([_WF.astype(np.int32), _HF.astype(np.int32), _WP1, _ABASE])
    rxy = reference_points.reshape(B, S, 8)

    value, gidx, w0, w1, w2, w3 = _proj(
        src, pos, rxy, Wv, bv.reshape(1, -1), Wcat, bcat, bd,
        jnp.asarray(ex), jnp.asarray(ey), jnp.asarray(cf),
        jnp.asarray(ci.astype(np.int32)))

    table = _build_anchor_table(value)

    def to4d(a):
        a = a.reshape(NW, NSUP, IR, 128)
        return jnp.pad(a, ((0, 0), (0, 0), (0, IRP - IR), (0, 0)))

    w_h = jnp.concatenate([w.reshape(N_OUT, 16) for w in (w0, w1, w2, w3)],
                          axis=1)
    attn_out = _sc_sample(table, to4d(gidx), w_h)

    ao = attn_out.reshape(B * S, D_MODEL)
    out = _tail(ao, src.reshape(B * S, D_MODEL), Wout, bout,
                g1, be1, W1, bf1, W2, bf2, g2, be2)
    return out.reshape(B, S, D_MODEL)
